# Initial kernel scaffold; baseline (speedup 1.0000x reference)
import jax
import jax.numpy as jnp
from jax.experimental import pallas as pl


def kernel(x, edge_index, weight, W1, b1, W2, b2, W3, b3, Wl1, bl1, Wl2, bl2):
    raise NotImplementedError("write your pallas kernel here")


# speedup vs baseline: 28.3373x; 28.3373x over previous
import functools

import jax
import jax.numpy as jnp
from jax import lax
from jax.experimental import pallas as pl
from jax.experimental.pallas import tpu as pltpu
from jax.experimental.pallas import tpu_sc as plsc

_N = 10000
_E = 320000
_D = 128
_H = 16

_NCOL = 10240
_NW = 32
_FPT = 4
_NSH = _NW // _FPT
_EPS = _E // _NSH
_CH = 4000
_EPT = _E // _NW
_CHD = 2000

_mesh = plsc.VectorSubcoreMesh(core_axis_name="c", subcore_axis_name="s")
_sc_params = pltpu.CompilerParams(needs_layout_passes=False)


@functools.partial(
    pl.kernel,
    mesh=_mesh,
    out_type=jax.ShapeDtypeStruct((_NW * _FPT * _NCOL,), jnp.float32),
    scratch_types=[
        pltpu.VMEM((_CH,), jnp.int32),
        pltpu.VMEM((_CH,), jnp.int32),
        pltpu.VMEM((_CH,), jnp.float32),
        pltpu.VMEM((_NCOL,), jnp.float32),
        pltpu.VMEM((_NCOL,), jnp.float32),
        pltpu.VMEM((_NCOL,), jnp.float32),
        pltpu.VMEM((_NCOL,), jnp.float32),
        pltpu.VMEM((_NCOL,), jnp.float32),
        pltpu.VMEM((_NCOL,), jnp.float32),
        pltpu.VMEM((_NCOL,), jnp.float32),
        pltpu.VMEM((_NCOL,), jnp.float32),
    ],
    compiler_params=_sc_params,
)
def _edge_accumulate(mt_hbm, r_hbm, c_hbm, w_hbm, out_hbm,
                     r_v, c_v, w_v, m0, m1, m2, m3, a0, a1, a2, a3):
    cid = lax.axis_index("c")
    sid = lax.axis_index("s")
    wid = cid * 16 + sid
    shard = wid // _FPT
    g = wid % _FPT

    mcols = [m0, m1, m2, m3]
    acols = [a0, a1, a2, a3]
    for q in range(_FPT):
        f = g * _FPT + q
        pltpu.sync_copy(mt_hbm.at[pl.ds(f * _NCOL, _NCOL)], mcols[q])

    zero = jnp.zeros((16,), jnp.float32)

    def _z(i, carry):
        for q in range(_FPT):
            acols[q][pl.ds(i * 16, 16)] = zero
        return carry

    lax.fori_loop(0, _NCOL // 16, _z, 0)

    ebase = shard * _EPS

    def _chunk(ci, carry):
        base = ebase + ci * _CH
        pltpu.sync_copy(r_hbm.at[pl.ds(base, _CH)], r_v)
        pltpu.sync_copy(c_hbm.at[pl.ds(base, _CH)], c_v)
        pltpu.sync_copy(w_hbm.at[pl.ds(base, _CH)], w_v)

        def _batch(b, carry2):
            o = b * 16
            r16 = r_v[pl.ds(o, 16)]
            c16 = c_v[pl.ds(o, 16)]
            w16 = w_v[pl.ds(o, 16)]
            for q in range(_FPT):
                vals = plsc.load_gather(mcols[q], [r16]) * w16
                plsc.addupdate_scatter(acols[q], [c16], vals)
            return carry2

        lax.fori_loop(0, _CH // 16, _batch, 0)
        return carry

    lax.fori_loop(0, _EPS // _CH, _chunk, 0)

    for q in range(_FPT):
        pltpu.sync_copy(acols[q],
                        out_hbm.at[pl.ds((wid * _FPT + q) * _NCOL, _NCOL)])


@functools.partial(
    pl.kernel,
    mesh=_mesh,
    out_type=jax.ShapeDtypeStruct((_NW * _NCOL,), jnp.float32),
    scratch_types=[
        pltpu.VMEM((_CHD,), jnp.int32),
        pltpu.VMEM((_CHD,), jnp.float32),
        pltpu.VMEM((_NCOL,), jnp.float32),
    ],
    compiler_params=_sc_params,
)
def _degree_accumulate(c_hbm, w_hbm, out_hbm, c_v, w_v, deg_v):
    cid = lax.axis_index("c")
    sid = lax.axis_index("s")
    wid = cid * 16 + sid

    zero = jnp.zeros((16,), jnp.float32)

    def _z(i, carry):
        deg_v[pl.ds(i * 16, 16)] = zero
        return carry

    lax.fori_loop(0, _NCOL // 16, _z, 0)

    ebase = wid * _EPT

    def _chunk(ci, carry):
        base = ebase + ci * _CHD
        pltpu.sync_copy(c_hbm.at[pl.ds(base, _CHD)], c_v)
        pltpu.sync_copy(w_hbm.at[pl.ds(base, _CHD)], w_v)

        def _batch(b, carry2):
            o = b * 16
            c16 = c_v[pl.ds(o, 16)]
            w16 = w_v[pl.ds(o, 16)]
            plsc.addupdate_scatter(deg_v, [c16], w16)
            return carry2

        lax.fori_loop(0, _CHD // 16, _batch, 0)
        return carry

    lax.fori_loop(0, _EPT // _CHD, _chunk, 0)
    pltpu.sync_copy(deg_v, out_hbm.at[pl.ds(wid * _NCOL, _NCOL)])


def _tc_prep(degp, xt, W1t):
    def body(degp_ref, xt_ref, w1t_ref, dinv_ref, m1t_ref):
        deg = jnp.sum(degp_ref[...], axis=0, keepdims=True) + 1.0
        dinv = lax.rsqrt(deg)
        dinv_ref[...] = dinv
        xw = jnp.dot(w1t_ref[...], xt_ref[...],
                     preferred_element_type=jnp.float32)
        m1t_ref[...] = jnp.concatenate(
            [xw * dinv[:, :_N], jnp.zeros((_H, _NCOL - _N), jnp.float32)],
            axis=1)

    return pl.pallas_call(
        body,
        out_shape=[
            jax.ShapeDtypeStruct((1, _NCOL), jnp.float32),
            jax.ShapeDtypeStruct((_H, _NCOL), jnp.float32),
        ],
    )(degp, xt, W1t)


def _tc_combine(acc8, mt, dinv, b, Wt):
    def body(acc_ref, mt_ref, dinv_ref, b_ref, wt_ref, out_ref):
        a = jnp.sum(acc_ref[...], axis=0) + mt_ref[...]
        h = jnp.maximum(a * dinv_ref[...] + b_ref[...], 0.0)
        hw = jnp.dot(wt_ref[...], h,
                     preferred_element_type=jnp.float32) * dinv_ref[...]
        out_ref[...] = jnp.concatenate(
            [hw[:, :_N], jnp.zeros((_H, _NCOL - _N), jnp.float32)], axis=1)

    return pl.pallas_call(
        body,
        out_shape=jax.ShapeDtypeStruct((_H, _NCOL), jnp.float32),
    )(acc8, mt, dinv, b, Wt)


def _tc_head(acc8, mt, dinv, b3, Wl1t, bl1, Wl2, bl2):
    def body(acc_ref, mt_ref, dinv_ref, b3_ref, wl1t_ref, bl1_ref, wl2_ref,
             bl2_ref, choice_ref, value_ref):
        a = jnp.sum(acc_ref[...], axis=0) + mt_ref[...]
        h3 = jnp.maximum(a * dinv_ref[...] + b3_ref[...], 0.0)
        h4 = jnp.maximum(
            jnp.dot(wl1t_ref[...], h3, preferred_element_type=jnp.float32)
            + bl1_ref[...], 0.0)
        wl2 = wl2_ref[...]
        logits = jnp.sum(h4 * wl2, axis=0, keepdims=True) + bl2_ref[...]
        cols = lax.broadcasted_iota(jnp.int32, (1, _NCOL), 1)
        valid = cols < _N
        neg = jnp.full((1, _NCOL), -jnp.inf, jnp.float32)
        mx = jnp.max(jnp.where(valid, logits, neg))
        ex = jnp.where(valid, jnp.exp(logits - mx), 0.0)
        choice_ref[...] = ex / jnp.sum(ex)
        h4m = jnp.where(valid, h4, 0.0)
        vmean = jnp.sum(h4m, axis=1, keepdims=True) / float(_N)
        value_ref[...] = (jnp.sum(vmean * wl2, axis=0, keepdims=True)
                          + bl2_ref[...])

    return pl.pallas_call(
        body,
        out_shape=[
            jax.ShapeDtypeStruct((1, _NCOL), jnp.float32),
            jax.ShapeDtypeStruct((1, 1), jnp.float32),
        ],
    )(acc8, mt, dinv, b3, Wl1t, bl1, Wl2, bl2)


def kernel(x, edge_index, weight, W1, b1, W2, b2, W3, b3, Wl1, bl1, Wl2, bl2):
    r = edge_index[0].astype(jnp.int32)
    c = edge_index[1].astype(jnp.int32)
    w = weight.astype(jnp.float32)

    degp = _degree_accumulate(c, w).reshape(_NW, _NCOL)
    dinv, m1t = _tc_prep(degp, x.T, W1.T)

    def conv(mt):
        accf = _edge_accumulate(mt.reshape(-1), r, c, w)
        return accf.reshape(_NSH, _H, _NCOL)

    acc1 = conv(m1t)
    m2t = _tc_combine(acc1, m1t, dinv, b1.reshape(_H, 1), W2.T)
    acc2 = conv(m2t)
    m3t = _tc_combine(acc2, m2t, dinv, b2.reshape(_H, 1), W3.T)
    acc3 = conv(m3t)
    choice, value = _tc_head(acc3, m3t, dinv, b3.reshape(_H, 1), Wl1.T,
                             bl1.reshape(_H, 1), Wl2, bl2.reshape(1, 1))
    return choice[0, :_N], value.reshape(())


# speedup vs baseline: 35.3057x; 1.2459x over previous
import functools

import jax
import jax.numpy as jnp
from jax import lax
from jax.experimental import pallas as pl
from jax.experimental.pallas import tpu as pltpu
from jax.experimental.pallas import tpu_sc as plsc

_N = 10000
_E = 320000
_D = 128
_H = 16

_NCOL = 10240
_NW = 32
_FPT = 4
_NSH = _NW // _FPT
_EPS = _E // _NSH
_CH = 4000
_EPT = _E // _NW
_CHD = 2000

_mesh = plsc.VectorSubcoreMesh(core_axis_name="c", subcore_axis_name="s")
_sc_params = pltpu.CompilerParams(needs_layout_passes=False)


@functools.partial(
    pl.kernel,
    mesh=_mesh,
    out_type=jax.ShapeDtypeStruct((_NW * _FPT * _NCOL,), jnp.float32),
    scratch_types=[
        pltpu.VMEM((_CH,), jnp.int32),
        pltpu.VMEM((_CH,), jnp.int32),
        pltpu.VMEM((_CH,), jnp.float32),
        pltpu.VMEM((_NCOL,), jnp.float32),
        pltpu.VMEM((_NCOL,), jnp.float32),
        pltpu.VMEM((_NCOL,), jnp.float32),
        pltpu.VMEM((_NCOL,), jnp.float32),
        pltpu.VMEM((_NCOL,), jnp.float32),
        pltpu.VMEM((_NCOL,), jnp.float32),
        pltpu.VMEM((_NCOL,), jnp.float32),
        pltpu.VMEM((_NCOL,), jnp.float32),
    ],
    compiler_params=_sc_params,
)
def _edge_accumulate(mt_hbm, r_hbm, c_hbm, w_hbm, out_hbm,
                     r_v, c_v, w_v, m0, m1, m2, m3, a0, a1, a2, a3):
    cid = lax.axis_index("c")
    sid = lax.axis_index("s")
    wid = cid * 16 + sid
    shard = wid // _FPT
    g = wid % _FPT

    mcols = [m0, m1, m2, m3]
    acols = [a0, a1, a2, a3]
    for q in range(_FPT):
        f = g * _FPT + q
        pltpu.sync_copy(mt_hbm.at[pl.ds(f * _NCOL, _NCOL)], mcols[q])

    zero = jnp.zeros((16,), jnp.float32)

    def _z(i, carry):
        for q in range(_FPT):
            acols[q][pl.ds(i * 16, 16)] = zero
        return carry

    lax.fori_loop(0, _NCOL // 16, _z, 0)

    ebase = shard * _EPS

    def _chunk(ci, carry):
        base = ebase + ci * _CH
        pltpu.sync_copy(r_hbm.at[pl.ds(base, _CH)], r_v)
        pltpu.sync_copy(c_hbm.at[pl.ds(base, _CH)], c_v)
        pltpu.sync_copy(w_hbm.at[pl.ds(base, _CH)], w_v)

        def _batch(b, carry2):
            o = b * 32
            r16a = r_v[pl.ds(o, 16)]
            c16a = c_v[pl.ds(o, 16)]
            w16a = w_v[pl.ds(o, 16)]
            r16b = r_v[pl.ds(o + 16, 16)]
            c16b = c_v[pl.ds(o + 16, 16)]
            w16b = w_v[pl.ds(o + 16, 16)]
            for q in range(_FPT):
                va = plsc.load_gather(mcols[q], [r16a]) * w16a
                vb = plsc.load_gather(mcols[q], [r16b]) * w16b
                plsc.addupdate_scatter(acols[q], [c16a], va)
                plsc.addupdate_scatter(acols[q], [c16b], vb)
            return carry2

        lax.fori_loop(0, _CH // 32, _batch, 0)
        return carry

    lax.fori_loop(0, _EPS // _CH, _chunk, 0)

    for q in range(_FPT):
        pltpu.sync_copy(acols[q],
                        out_hbm.at[pl.ds((wid * _FPT + q) * _NCOL, _NCOL)])


@functools.partial(
    pl.kernel,
    mesh=_mesh,
    out_type=jax.ShapeDtypeStruct((_NW * _NCOL,), jnp.float32),
    scratch_types=[
        pltpu.VMEM((_CHD,), jnp.int32),
        pltpu.VMEM((_CHD,), jnp.float32),
        pltpu.VMEM((_NCOL,), jnp.float32),
    ],
    compiler_params=_sc_params,
)
def _degree_accumulate(c_hbm, w_hbm, out_hbm, c_v, w_v, deg_v):
    cid = lax.axis_index("c")
    sid = lax.axis_index("s")
    wid = cid * 16 + sid

    zero = jnp.zeros((16,), jnp.float32)

    def _z(i, carry):
        deg_v[pl.ds(i * 16, 16)] = zero
        return carry

    lax.fori_loop(0, _NCOL // 16, _z, 0)

    ebase = wid * _EPT

    def _chunk(ci, carry):
        base = ebase + ci * _CHD
        pltpu.sync_copy(c_hbm.at[pl.ds(base, _CHD)], c_v)
        pltpu.sync_copy(w_hbm.at[pl.ds(base, _CHD)], w_v)

        def _batch(b, carry2):
            o = b * 16
            c16 = c_v[pl.ds(o, 16)]
            w16 = w_v[pl.ds(o, 16)]
            plsc.addupdate_scatter(deg_v, [c16], w16)
            return carry2

        lax.fori_loop(0, _CHD // 16, _batch, 0)
        return carry

    lax.fori_loop(0, _EPT // _CHD, _chunk, 0)
    pltpu.sync_copy(deg_v, out_hbm.at[pl.ds(wid * _NCOL, _NCOL)])


def _tc_prep(degp, xt, W1t):
    def body(degp_ref, xt_ref, w1t_ref, dinv_ref, m1t_ref):
        deg = jnp.sum(degp_ref[...], axis=0, keepdims=True) + 1.0
        dinv = lax.rsqrt(deg)
        dinv_ref[...] = dinv
        xw = jnp.dot(w1t_ref[...], xt_ref[...],
                     preferred_element_type=jnp.float32)
        m1t_ref[...] = jnp.concatenate(
            [xw * dinv[:, :_N], jnp.zeros((_H, _NCOL - _N), jnp.float32)],
            axis=1)

    return pl.pallas_call(
        body,
        out_shape=[
            jax.ShapeDtypeStruct((1, _NCOL), jnp.float32),
            jax.ShapeDtypeStruct((_H, _NCOL), jnp.float32),
        ],
    )(degp, xt, W1t)


def _tc_combine(acc8, mt, dinv, b, Wt):
    def body(acc_ref, mt_ref, dinv_ref, b_ref, wt_ref, out_ref):
        a = jnp.sum(acc_ref[...], axis=0) + mt_ref[...]
        h = jnp.maximum(a * dinv_ref[...] + b_ref[...], 0.0)
        hw = jnp.dot(wt_ref[...], h,
                     preferred_element_type=jnp.float32) * dinv_ref[...]
        out_ref[...] = jnp.concatenate(
            [hw[:, :_N], jnp.zeros((_H, _NCOL - _N), jnp.float32)], axis=1)

    return pl.pallas_call(
        body,
        out_shape=jax.ShapeDtypeStruct((_H, _NCOL), jnp.float32),
    )(acc8, mt, dinv, b, Wt)


def _tc_head(acc8, mt, dinv, b3, Wl1t, bl1, Wl2, bl2):
    def body(acc_ref, mt_ref, dinv_ref, b3_ref, wl1t_ref, bl1_ref, wl2_ref,
             bl2_ref, choice_ref, value_ref):
        a = jnp.sum(acc_ref[...], axis=0) + mt_ref[...]
        h3 = jnp.maximum(a * dinv_ref[...] + b3_ref[...], 0.0)
        h4 = jnp.maximum(
            jnp.dot(wl1t_ref[...], h3, preferred_element_type=jnp.float32)
            + bl1_ref[...], 0.0)
        wl2 = wl2_ref[...]
        logits = jnp.sum(h4 * wl2, axis=0, keepdims=True) + bl2_ref[...]
        cols = lax.broadcasted_iota(jnp.int32, (1, _NCOL), 1)
        valid = cols < _N
        neg = jnp.full((1, _NCOL), -jnp.inf, jnp.float32)
        mx = jnp.max(jnp.where(valid, logits, neg))
        ex = jnp.where(valid, jnp.exp(logits - mx), 0.0)
        choice_ref[...] = ex / jnp.sum(ex)
        h4m = jnp.where(valid, h4, 0.0)
        vmean = jnp.sum(h4m, axis=1, keepdims=True) / float(_N)
        value_ref[...] = (jnp.sum(vmean * wl2, axis=0, keepdims=True)
                          + bl2_ref[...])

    return pl.pallas_call(
        body,
        out_shape=[
            jax.ShapeDtypeStruct((1, _NCOL), jnp.float32),
            jax.ShapeDtypeStruct((1, 1), jnp.float32),
        ],
    )(acc8, mt, dinv, b3, Wl1t, bl1, Wl2, bl2)


def kernel(x, edge_index, weight, W1, b1, W2, b2, W3, b3, Wl1, bl1, Wl2, bl2):
    r = edge_index[0].astype(jnp.int32)
    c = edge_index[1].astype(jnp.int32)
    w = weight.astype(jnp.float32)

    degp = _degree_accumulate(c, w).reshape(_NW, _NCOL)
    dinv, m1t = _tc_prep(degp, x.T, W1.T)

    def conv(mt):
        accf = _edge_accumulate(mt.reshape(-1), r, c, w)
        return accf.reshape(_NSH, _H, _NCOL)

    acc1 = conv(m1t)
    m2t = _tc_combine(acc1, m1t, dinv, b1.reshape(_H, 1), W2.T)
    acc2 = conv(m2t)
    m3t = _tc_combine(acc2, m2t, dinv, b2.reshape(_H, 1), W3.T)
    acc3 = conv(m3t)
    choice, value = _tc_head(acc3, m3t, dinv, b3.reshape(_H, 1), Wl1.T,
                             bl1.reshape(_H, 1), Wl2, bl2.reshape(1, 1))
    return choice[0, :_N], value.reshape(())


# speedup vs baseline: 40.5836x; 1.1495x over previous
import functools

import jax
import jax.numpy as jnp
from jax import lax
from jax.experimental import pallas as pl
from jax.experimental.pallas import tpu as pltpu
from jax.experimental.pallas import tpu_sc as plsc

_N = 10000
_E = 320000
_D = 128
_H = 16

_NCOL = 10240
_NW = 32
_FPT = 4
_NSH = _NW // _FPT
_EPS = _E // _NSH
_CH = 4000
_EPT = _E // _NW
_CHD = 2000

_mesh = plsc.VectorSubcoreMesh(core_axis_name="c", subcore_axis_name="s")
_sc_params = pltpu.CompilerParams(needs_layout_passes=False)


@functools.partial(
    pl.kernel,
    mesh=_mesh,
    out_type=jax.ShapeDtypeStruct((_NW * _FPT * _NCOL,), jnp.float32),
    scratch_types=[
        pltpu.VMEM((_CH,), jnp.int32),
        pltpu.VMEM((_CH,), jnp.int32),
        pltpu.VMEM((_CH,), jnp.float32),
        pltpu.VMEM((_NCOL,), jnp.float32),
        pltpu.VMEM((_NCOL,), jnp.float32),
        pltpu.VMEM((_NCOL,), jnp.float32),
        pltpu.VMEM((_NCOL,), jnp.float32),
        pltpu.VMEM((_NCOL,), jnp.float32),
        pltpu.VMEM((_NCOL,), jnp.float32),
        pltpu.VMEM((_NCOL,), jnp.float32),
        pltpu.VMEM((_NCOL,), jnp.float32),
    ],
    compiler_params=_sc_params,
)
def _edge_accumulate(mt_hbm, r_hbm, c_hbm, w_hbm, out_hbm,
                     r_v, c_v, w_v, m0, m1, m2, m3, a0, a1, a2, a3):
    cid = lax.axis_index("c")
    sid = lax.axis_index("s")
    wid = cid * 16 + sid
    shard = wid // _FPT
    g = wid % _FPT

    mcols = [m0, m1, m2, m3]
    acols = [a0, a1, a2, a3]
    for q in range(_FPT):
        f = g * _FPT + q
        pltpu.sync_copy(mt_hbm.at[pl.ds(f * _NCOL, _NCOL)], mcols[q])

    zero = jnp.zeros((16,), jnp.float32)

    def _z(i, carry):
        for q in range(_FPT):
            acols[q][pl.ds(i * 16, 16)] = zero
        return carry

    lax.fori_loop(0, _NCOL // 16, _z, 0)

    ebase = shard * _EPS

    def _chunk(ci, carry):
        base = ebase + ci * _CH
        pltpu.sync_copy(r_hbm.at[pl.ds(base, _CH)], r_v)
        pltpu.sync_copy(c_hbm.at[pl.ds(base, _CH)], c_v)
        pltpu.sync_copy(w_hbm.at[pl.ds(base, _CH)], w_v)

        def _batch(b, carry2):
            o = b * 64
            idx = [(r_v[pl.ds(o + 16 * u, 16)], c_v[pl.ds(o + 16 * u, 16)],
                    w_v[pl.ds(o + 16 * u, 16)]) for u in range(4)]
            for q in range(_FPT):
                vals = [plsc.load_gather(mcols[q], [r16]) * w16
                        for (r16, _, w16) in idx]
                for (_, c16, _), v in zip(idx, vals):
                    plsc.addupdate_scatter(acols[q], [c16], v)
            return carry2

        lax.fori_loop(0, _CH // 64, _batch, 0)
        return carry

    lax.fori_loop(0, _EPS // _CH, _chunk, 0)

    for q in range(_FPT):
        pltpu.sync_copy(acols[q],
                        out_hbm.at[pl.ds((wid * _FPT + q) * _NCOL, _NCOL)])


@functools.partial(
    pl.kernel,
    mesh=_mesh,
    out_type=jax.ShapeDtypeStruct((_NW * _NCOL,), jnp.float32),
    scratch_types=[
        pltpu.VMEM((_CHD,), jnp.int32),
        pltpu.VMEM((_CHD,), jnp.float32),
        pltpu.VMEM((_NCOL,), jnp.float32),
    ],
    compiler_params=_sc_params,
)
def _degree_accumulate(c_hbm, w_hbm, out_hbm, c_v, w_v, deg_v):
    cid = lax.axis_index("c")
    sid = lax.axis_index("s")
    wid = cid * 16 + sid

    zero = jnp.zeros((16,), jnp.float32)

    def _z(i, carry):
        deg_v[pl.ds(i * 16, 16)] = zero
        return carry

    lax.fori_loop(0, _NCOL // 16, _z, 0)

    ebase = wid * _EPT

    def _chunk(ci, carry):
        base = ebase + ci * _CHD
        pltpu.sync_copy(c_hbm.at[pl.ds(base, _CHD)], c_v)
        pltpu.sync_copy(w_hbm.at[pl.ds(base, _CHD)], w_v)

        def _batch(b, carry2):
            o = b * 16
            c16 = c_v[pl.ds(o, 16)]
            w16 = w_v[pl.ds(o, 16)]
            plsc.addupdate_scatter(deg_v, [c16], w16)
            return carry2

        lax.fori_loop(0, _CHD // 16, _batch, 0)
        return carry

    lax.fori_loop(0, _EPT // _CHD, _chunk, 0)
    pltpu.sync_copy(deg_v, out_hbm.at[pl.ds(wid * _NCOL, _NCOL)])


def _tc_prep(degp, xt, W1t):
    def body(degp_ref, xt_ref, w1t_ref, dinv_ref, m1t_ref):
        deg = jnp.sum(degp_ref[...], axis=0, keepdims=True) + 1.0
        dinv = lax.rsqrt(deg)
        dinv_ref[...] = dinv
        xw = jnp.dot(w1t_ref[...], xt_ref[...],
                     preferred_element_type=jnp.float32)
        m1t_ref[...] = jnp.concatenate(
            [xw * dinv[:, :_N], jnp.zeros((_H, _NCOL - _N), jnp.float32)],
            axis=1)

    return pl.pallas_call(
        body,
        out_shape=[
            jax.ShapeDtypeStruct((1, _NCOL), jnp.float32),
            jax.ShapeDtypeStruct((_H, _NCOL), jnp.float32),
        ],
    )(degp, xt, W1t)


def _tc_combine(acc8, mt, dinv, b, Wt):
    def body(acc_ref, mt_ref, dinv_ref, b_ref, wt_ref, out_ref):
        a = jnp.sum(acc_ref[...], axis=0) + mt_ref[...]
        h = jnp.maximum(a * dinv_ref[...] + b_ref[...], 0.0)
        hw = jnp.dot(wt_ref[...], h,
                     preferred_element_type=jnp.float32) * dinv_ref[...]
        out_ref[...] = jnp.concatenate(
            [hw[:, :_N], jnp.zeros((_H, _NCOL - _N), jnp.float32)], axis=1)

    return pl.pallas_call(
        body,
        out_shape=jax.ShapeDtypeStruct((_H, _NCOL), jnp.float32),
    )(acc8, mt, dinv, b, Wt)


def _tc_head(acc8, mt, dinv, b3, Wl1t, bl1, Wl2, bl2):
    def body(acc_ref, mt_ref, dinv_ref, b3_ref, wl1t_ref, bl1_ref, wl2_ref,
             bl2_ref, choice_ref, value_ref):
        a = jnp.sum(acc_ref[...], axis=0) + mt_ref[...]
        h3 = jnp.maximum(a * dinv_ref[...] + b3_ref[...], 0.0)
        h4 = jnp.maximum(
            jnp.dot(wl1t_ref[...], h3, preferred_element_type=jnp.float32)
            + bl1_ref[...], 0.0)
        wl2 = wl2_ref[...]
        logits = jnp.sum(h4 * wl2, axis=0, keepdims=True) + bl2_ref[...]
        cols = lax.broadcasted_iota(jnp.int32, (1, _NCOL), 1)
        valid = cols < _N
        neg = jnp.full((1, _NCOL), -jnp.inf, jnp.float32)
        mx = jnp.max(jnp.where(valid, logits, neg))
        ex = jnp.where(valid, jnp.exp(logits - mx), 0.0)
        choice_ref[...] = ex / jnp.sum(ex)
        h4m = jnp.where(valid, h4, 0.0)
        vmean = jnp.sum(h4m, axis=1, keepdims=True) / float(_N)
        value_ref[...] = (jnp.sum(vmean * wl2, axis=0, keepdims=True)
                          + bl2_ref[...])

    return pl.pallas_call(
        body,
        out_shape=[
            jax.ShapeDtypeStruct((1, _NCOL), jnp.float32),
            jax.ShapeDtypeStruct((1, 1), jnp.float32),
        ],
    )(acc8, mt, dinv, b3, Wl1t, bl1, Wl2, bl2)


def kernel(x, edge_index, weight, W1, b1, W2, b2, W3, b3, Wl1, bl1, Wl2, bl2):
    r = edge_index[0].astype(jnp.int32)
    c = edge_index[1].astype(jnp.int32)
    w = weight.astype(jnp.float32)

    degp = _degree_accumulate(c, w).reshape(_NW, _NCOL)
    dinv, m1t = _tc_prep(degp, x.T, W1.T)

    def conv(mt):
        accf = _edge_accumulate(mt.reshape(-1), r, c, w)
        return accf.reshape(_NSH, _H, _NCOL)

    acc1 = conv(m1t)
    m2t = _tc_combine(acc1, m1t, dinv, b1.reshape(_H, 1), W2.T)
    acc2 = conv(m2t)
    m3t = _tc_combine(acc2, m2t, dinv, b2.reshape(_H, 1), W3.T)
    acc3 = conv(m3t)
    choice, value = _tc_head(acc3, m3t, dinv, b3.reshape(_H, 1), Wl1.T,
                             bl1.reshape(_H, 1), Wl2, bl2.reshape(1, 1))
    return choice[0, :_N], value.reshape(())


# speedup vs baseline: 42.2356x; 1.0407x over previous
import functools

import jax
import jax.numpy as jnp
from jax import lax
from jax.experimental import pallas as pl
from jax.experimental.pallas import tpu as pltpu
from jax.experimental.pallas import tpu_sc as plsc

_N = 10000
_E = 320000
_D = 128
_H = 16

_NCOL = 10240
_NW = 32
_FPT = 4
_NSH = _NW // _FPT
_EPS = _E // _NSH
_CH = 4000
_EPT = _E // _NW
_CHD = 2000

_mesh = plsc.VectorSubcoreMesh(core_axis_name="c", subcore_axis_name="s")
_sc_params = pltpu.CompilerParams(needs_layout_passes=False)


@functools.partial(
    pl.kernel,
    mesh=_mesh,
    out_type=jax.ShapeDtypeStruct((_NW * _FPT * _NCOL,), jnp.float32),
    scratch_types=[
        pltpu.VMEM((_CH,), jnp.int32),
        pltpu.VMEM((_CH,), jnp.int32),
        pltpu.VMEM((_CH,), jnp.float32),
        pltpu.VMEM((_NCOL,), jnp.float32),
        pltpu.VMEM((_NCOL,), jnp.float32),
        pltpu.VMEM((_NCOL,), jnp.float32),
        pltpu.VMEM((_NCOL,), jnp.float32),
        pltpu.VMEM((_NCOL,), jnp.float32),
        pltpu.VMEM((_NCOL,), jnp.float32),
        pltpu.VMEM((_NCOL,), jnp.float32),
        pltpu.VMEM((_NCOL,), jnp.float32),
    ],
    compiler_params=_sc_params,
)
def _edge_accumulate(mt_hbm, r_hbm, c_hbm, w_hbm, out_hbm,
                     r_v, c_v, w_v, m0, m1, m2, m3, a0, a1, a2, a3):
    cid = lax.axis_index("c")
    sid = lax.axis_index("s")
    wid = cid * 16 + sid
    shard = wid // _FPT
    g = wid % _FPT

    mcols = [m0, m1, m2, m3]
    acols = [a0, a1, a2, a3]
    for q in range(_FPT):
        f = g * _FPT + q
        pltpu.sync_copy(mt_hbm.at[pl.ds(f * _NCOL, _NCOL)], mcols[q])

    zero = jnp.zeros((16,), jnp.float32)

    def _z(i, carry):
        for q in range(_FPT):
            acols[q][pl.ds(i * 16, 16)] = zero
        return carry

    lax.fori_loop(0, _NCOL // 16, _z, 0)

    ebase = shard * _EPS

    def _chunk(ci, carry):
        base = ebase + ci * _CH
        pltpu.sync_copy(r_hbm.at[pl.ds(base, _CH)], r_v)
        pltpu.sync_copy(c_hbm.at[pl.ds(base, _CH)], c_v)
        pltpu.sync_copy(w_hbm.at[pl.ds(base, _CH)], w_v)

        def _batch(b, carry2):
            o = b * 64
            idx = [(r_v[pl.ds(o + 16 * u, 16)], c_v[pl.ds(o + 16 * u, 16)],
                    w_v[pl.ds(o + 16 * u, 16)]) for u in range(4)]
            vals = [[plsc.load_gather(mcols[q], [r16]) * w16
                     for q in range(_FPT)] for (r16, _, w16) in idx]
            for u in range(4):
                c16 = idx[u][1]
                for q in range(_FPT):
                    plsc.addupdate_scatter(acols[q], [c16], vals[u][q])
            return carry2

        lax.fori_loop(0, _CH // 64, _batch, 0)
        return carry

    lax.fori_loop(0, _EPS // _CH, _chunk, 0)

    for q in range(_FPT):
        pltpu.sync_copy(acols[q],
                        out_hbm.at[pl.ds((wid * _FPT + q) * _NCOL, _NCOL)])


@functools.partial(
    pl.kernel,
    mesh=_mesh,
    out_type=jax.ShapeDtypeStruct((_NW * _NCOL,), jnp.float32),
    scratch_types=[
        pltpu.VMEM((_CHD,), jnp.int32),
        pltpu.VMEM((_CHD,), jnp.float32),
        pltpu.VMEM((_NCOL,), jnp.float32),
    ],
    compiler_params=_sc_params,
)
def _degree_accumulate(c_hbm, w_hbm, out_hbm, c_v, w_v, deg_v):
    cid = lax.axis_index("c")
    sid = lax.axis_index("s")
    wid = cid * 16 + sid

    zero = jnp.zeros((16,), jnp.float32)

    def _z(i, carry):
        deg_v[pl.ds(i * 16, 16)] = zero
        return carry

    lax.fori_loop(0, _NCOL // 16, _z, 0)

    ebase = wid * _EPT

    def _chunk(ci, carry):
        base = ebase + ci * _CHD
        pltpu.sync_copy(c_hbm.at[pl.ds(base, _CHD)], c_v)
        pltpu.sync_copy(w_hbm.at[pl.ds(base, _CHD)], w_v)

        def _batch(b, carry2):
            o = b * 16
            c16 = c_v[pl.ds(o, 16)]
            w16 = w_v[pl.ds(o, 16)]
            plsc.addupdate_scatter(deg_v, [c16], w16)
            return carry2

        lax.fori_loop(0, _CHD // 16, _batch, 0)
        return carry

    lax.fori_loop(0, _EPT // _CHD, _chunk, 0)
    pltpu.sync_copy(deg_v, out_hbm.at[pl.ds(wid * _NCOL, _NCOL)])


def _tc_prep(degp, xt, W1t):
    def body(degp_ref, xt_ref, w1t_ref, dinv_ref, m1t_ref):
        deg = jnp.sum(degp_ref[...], axis=0, keepdims=True) + 1.0
        dinv = lax.rsqrt(deg)
        dinv_ref[...] = dinv
        xw = jnp.dot(w1t_ref[...], xt_ref[...],
                     preferred_element_type=jnp.float32)
        m1t_ref[...] = jnp.concatenate(
            [xw * dinv[:, :_N], jnp.zeros((_H, _NCOL - _N), jnp.float32)],
            axis=1)

    return pl.pallas_call(
        body,
        out_shape=[
            jax.ShapeDtypeStruct((1, _NCOL), jnp.float32),
            jax.ShapeDtypeStruct((_H, _NCOL), jnp.float32),
        ],
    )(degp, xt, W1t)


def _tc_combine(acc8, mt, dinv, b, Wt):
    def body(acc_ref, mt_ref, dinv_ref, b_ref, wt_ref, out_ref):
        a = jnp.sum(acc_ref[...], axis=0) + mt_ref[...]
        h = jnp.maximum(a * dinv_ref[...] + b_ref[...], 0.0)
        hw = jnp.dot(wt_ref[...], h,
                     preferred_element_type=jnp.float32) * dinv_ref[...]
        out_ref[...] = jnp.concatenate(
            [hw[:, :_N], jnp.zeros((_H, _NCOL - _N), jnp.float32)], axis=1)

    return pl.pallas_call(
        body,
        out_shape=jax.ShapeDtypeStruct((_H, _NCOL), jnp.float32),
    )(acc8, mt, dinv, b, Wt)


def _tc_head(acc8, mt, dinv, b3, Wl1t, bl1, Wl2, bl2):
    def body(acc_ref, mt_ref, dinv_ref, b3_ref, wl1t_ref, bl1_ref, wl2_ref,
             bl2_ref, choice_ref, value_ref):
        a = jnp.sum(acc_ref[...], axis=0) + mt_ref[...]
        h3 = jnp.maximum(a * dinv_ref[...] + b3_ref[...], 0.0)
        h4 = jnp.maximum(
            jnp.dot(wl1t_ref[...], h3, preferred_element_type=jnp.float32)
            + bl1_ref[...], 0.0)
        wl2 = wl2_ref[...]
        logits = jnp.sum(h4 * wl2, axis=0, keepdims=True) + bl2_ref[...]
        cols = lax.broadcasted_iota(jnp.int32, (1, _NCOL), 1)
        valid = cols < _N
        neg = jnp.full((1, _NCOL), -jnp.inf, jnp.float32)
        mx = jnp.max(jnp.where(valid, logits, neg))
        ex = jnp.where(valid, jnp.exp(logits - mx), 0.0)
        choice_ref[...] = ex / jnp.sum(ex)
        h4m = jnp.where(valid, h4, 0.0)
        vmean = jnp.sum(h4m, axis=1, keepdims=True) / float(_N)
        value_ref[...] = (jnp.sum(vmean * wl2, axis=0, keepdims=True)
                          + bl2_ref[...])

    return pl.pallas_call(
        body,
        out_shape=[
            jax.ShapeDtypeStruct((1, _NCOL), jnp.float32),
            jax.ShapeDtypeStruct((1, 1), jnp.float32),
        ],
    )(acc8, mt, dinv, b3, Wl1t, bl1, Wl2, bl2)


def kernel(x, edge_index, weight, W1, b1, W2, b2, W3, b3, Wl1, bl1, Wl2, bl2):
    r = edge_index[0].astype(jnp.int32)
    c = edge_index[1].astype(jnp.int32)
    w = weight.astype(jnp.float32)

    degp = _degree_accumulate(c, w).reshape(_NW, _NCOL)
    dinv, m1t = _tc_prep(degp, x.T, W1.T)

    def conv(mt):
        accf = _edge_accumulate(mt.reshape(-1), r, c, w)
        return accf.reshape(_NSH, _H, _NCOL)

    acc1 = conv(m1t)
    m2t = _tc_combine(acc1, m1t, dinv, b1.reshape(_H, 1), W2.T)
    acc2 = conv(m2t)
    m3t = _tc_combine(acc2, m2t, dinv, b2.reshape(_H, 1), W3.T)
    acc3 = conv(m3t)
    choice, value = _tc_head(acc3, m3t, dinv, b3.reshape(_H, 1), Wl1.T,
                             bl1.reshape(_H, 1), Wl2, bl2.reshape(1, 1))
    return choice[0, :_N], value.reshape(())


# speedup vs baseline: 46.5088x; 1.1012x over previous
import functools

import jax
import jax.numpy as jnp
from jax import lax
from jax.experimental import pallas as pl
from jax.experimental.pallas import tpu as pltpu
from jax.experimental.pallas import tpu_sc as plsc

_N = 10000
_E = 320000
_D = 128
_H = 16

_NCOL = 10240
_NW = 32
_FPT = 4
_NSH = _NW // _FPT
_EPS = _E // _NSH
_CH = 8000
_EPT = _E // _NW
_CHD = 2000

_mesh = plsc.VectorSubcoreMesh(core_axis_name="c", subcore_axis_name="s")
_sc_params = pltpu.CompilerParams(needs_layout_passes=False)


@functools.partial(
    pl.kernel,
    mesh=_mesh,
    out_type=jax.ShapeDtypeStruct((_NW * _FPT * _NCOL,), jnp.float32),
    scratch_types=[
        pltpu.VMEM((_CH,), jnp.int32),
        pltpu.VMEM((_CH,), jnp.int32),
        pltpu.VMEM((_CH,), jnp.float32),
        pltpu.VMEM((_NCOL,), jnp.float32),
        pltpu.VMEM((_NCOL,), jnp.float32),
        pltpu.VMEM((_NCOL,), jnp.float32),
        pltpu.VMEM((_NCOL,), jnp.float32),
        pltpu.VMEM((_NCOL,), jnp.float32),
        pltpu.VMEM((_NCOL,), jnp.float32),
        pltpu.VMEM((_NCOL,), jnp.float32),
        pltpu.VMEM((_NCOL,), jnp.float32),
    ],
    compiler_params=_sc_params,
)
def _edge_accumulate(mt_hbm, r_hbm, c_hbm, w_hbm, out_hbm,
                     r_v, c_v, w_v, m0, m1, m2, m3, a0, a1, a2, a3):
    cid = lax.axis_index("c")
    sid = lax.axis_index("s")
    wid = cid * 16 + sid
    shard = wid // _FPT
    g = wid % _FPT

    mcols = [m0, m1, m2, m3]
    acols = [a0, a1, a2, a3]
    for q in range(_FPT):
        f = g * _FPT + q
        pltpu.sync_copy(mt_hbm.at[pl.ds(f * _NCOL, _NCOL)], mcols[q])

    zero = jnp.zeros((16,), jnp.float32)

    def _z(i, carry):
        for q in range(_FPT):
            acols[q][pl.ds(i * 16, 16)] = zero
        return carry

    lax.fori_loop(0, _NCOL // 16, _z, 0)

    ebase = shard * _EPS

    def _chunk(ci, carry):
        base = ebase + ci * _CH
        pltpu.sync_copy(r_hbm.at[pl.ds(base, _CH)], r_v)
        pltpu.sync_copy(c_hbm.at[pl.ds(base, _CH)], c_v)
        pltpu.sync_copy(w_hbm.at[pl.ds(base, _CH)], w_v)

        def _batch(b, carry2):
            o = b * 64
            idx = [(r_v[pl.ds(o + 16 * u, 16)], c_v[pl.ds(o + 16 * u, 16)],
                    w_v[pl.ds(o + 16 * u, 16)]) for u in range(4)]
            vals = [[plsc.load_gather(mcols[q], [r16]) * w16
                     for q in range(_FPT)] for (r16, _, w16) in idx]
            for u in range(4):
                c16 = idx[u][1]
                for q in range(_FPT):
                    plsc.addupdate_scatter(acols[q], [c16], vals[u][q])
            return carry2

        lax.fori_loop(0, _CH // 64, _batch, 0)
        return carry

    lax.fori_loop(0, _EPS // _CH, _chunk, 0)

    for q in range(_FPT):
        pltpu.sync_copy(acols[q],
                        out_hbm.at[pl.ds((wid * _FPT + q) * _NCOL, _NCOL)])


@functools.partial(
    pl.kernel,
    mesh=_mesh,
    out_type=jax.ShapeDtypeStruct((_NW * _NCOL,), jnp.float32),
    scratch_types=[
        pltpu.VMEM((_CHD,), jnp.int32),
        pltpu.VMEM((_CHD,), jnp.float32),
        pltpu.VMEM((_NCOL,), jnp.float32),
    ],
    compiler_params=_sc_params,
)
def _degree_accumulate(c_hbm, w_hbm, out_hbm, c_v, w_v, deg_v):
    cid = lax.axis_index("c")
    sid = lax.axis_index("s")
    wid = cid * 16 + sid

    zero = jnp.zeros((16,), jnp.float32)

    def _z(i, carry):
        deg_v[pl.ds(i * 16, 16)] = zero
        return carry

    lax.fori_loop(0, _NCOL // 16, _z, 0)

    ebase = wid * _EPT

    def _chunk(ci, carry):
        base = ebase + ci * _CHD
        pltpu.sync_copy(c_hbm.at[pl.ds(base, _CHD)], c_v)
        pltpu.sync_copy(w_hbm.at[pl.ds(base, _CHD)], w_v)

        def _batch(b, carry2):
            o = b * 16
            c16 = c_v[pl.ds(o, 16)]
            w16 = w_v[pl.ds(o, 16)]
            plsc.addupdate_scatter(deg_v, [c16], w16)
            return carry2

        lax.fori_loop(0, _CHD // 16, _batch, 0)
        return carry

    lax.fori_loop(0, _EPT // _CHD, _chunk, 0)
    pltpu.sync_copy(deg_v, out_hbm.at[pl.ds(wid * _NCOL, _NCOL)])


def _tc_prep(degp, xt, W1t):
    def body(degp_ref, xt_ref, w1t_ref, dinv_ref, m1t_ref):
        deg = jnp.sum(degp_ref[...], axis=0, keepdims=True) + 1.0
        dinv = lax.rsqrt(deg)
        dinv_ref[...] = dinv
        xw = jnp.dot(w1t_ref[...], xt_ref[...],
                     preferred_element_type=jnp.float32)
        m1t_ref[...] = jnp.concatenate(
            [xw * dinv[:, :_N], jnp.zeros((_H, _NCOL - _N), jnp.float32)],
            axis=1)

    return pl.pallas_call(
        body,
        out_shape=[
            jax.ShapeDtypeStruct((1, _NCOL), jnp.float32),
            jax.ShapeDtypeStruct((_H, _NCOL), jnp.float32),
        ],
    )(degp, xt, W1t)


def _tc_combine(acc8, mt, dinv, b, Wt):
    def body(acc_ref, mt_ref, dinv_ref, b_ref, wt_ref, out_ref):
        a = jnp.sum(acc_ref[...], axis=0) + mt_ref[...]
        h = jnp.maximum(a * dinv_ref[...] + b_ref[...], 0.0)
        hw = jnp.dot(wt_ref[...], h,
                     preferred_element_type=jnp.float32) * dinv_ref[...]
        out_ref[...] = jnp.concatenate(
            [hw[:, :_N], jnp.zeros((_H, _NCOL - _N), jnp.float32)], axis=1)

    return pl.pallas_call(
        body,
        out_shape=jax.ShapeDtypeStruct((_H, _NCOL), jnp.float32),
    )(acc8, mt, dinv, b, Wt)


def _tc_head(acc8, mt, dinv, b3, Wl1t, bl1, Wl2, bl2):
    def body(acc_ref, mt_ref, dinv_ref, b3_ref, wl1t_ref, bl1_ref, wl2_ref,
             bl2_ref, choice_ref, value_ref):
        a = jnp.sum(acc_ref[...], axis=0) + mt_ref[...]
        h3 = jnp.maximum(a * dinv_ref[...] + b3_ref[...], 0.0)
        h4 = jnp.maximum(
            jnp.dot(wl1t_ref[...], h3, preferred_element_type=jnp.float32)
            + bl1_ref[...], 0.0)
        wl2 = wl2_ref[...]
        logits = jnp.sum(h4 * wl2, axis=0, keepdims=True) + bl2_ref[...]
        cols = lax.broadcasted_iota(jnp.int32, (1, _NCOL), 1)
        valid = cols < _N
        neg = jnp.full((1, _NCOL), -jnp.inf, jnp.float32)
        mx = jnp.max(jnp.where(valid, logits, neg))
        ex = jnp.where(valid, jnp.exp(logits - mx), 0.0)
        choice_ref[...] = ex / jnp.sum(ex)
        h4m = jnp.where(valid, h4, 0.0)
        vmean = jnp.sum(h4m, axis=1, keepdims=True) / float(_N)
        value_ref[...] = (jnp.sum(vmean * wl2, axis=0, keepdims=True)
                          + bl2_ref[...])

    return pl.pallas_call(
        body,
        out_shape=[
            jax.ShapeDtypeStruct((1, _NCOL), jnp.float32),
            jax.ShapeDtypeStruct((1, 1), jnp.float32),
        ],
    )(acc8, mt, dinv, b3, Wl1t, bl1, Wl2, bl2)


def kernel(x, edge_index, weight, W1, b1, W2, b2, W3, b3, Wl1, bl1, Wl2, bl2):
    r = edge_index[0].astype(jnp.int32)
    c = edge_index[1].astype(jnp.int32)
    w = weight.astype(jnp.float32)

    degp = _degree_accumulate(c, w).reshape(_NW, _NCOL)
    dinv, m1t = _tc_prep(degp, x.T, W1.T)

    def conv(mt):
        accf = _edge_accumulate(mt.reshape(-1), r, c, w)
        return accf.reshape(_NSH, _H, _NCOL)

    acc1 = conv(m1t)
    m2t = _tc_combine(acc1, m1t, dinv, b1.reshape(_H, 1), W2.T)
    acc2 = conv(m2t)
    m3t = _tc_combine(acc2, m2t, dinv, b2.reshape(_H, 1), W3.T)
    acc3 = conv(m3t)
    choice, value = _tc_head(acc3, m3t, dinv, b3.reshape(_H, 1), Wl1.T,
                             bl1.reshape(_H, 1), Wl2, bl2.reshape(1, 1))
    return choice[0, :_N], value.reshape(())
